# baseline (speedup 1.0000x reference)
import functools

import jax
import jax.numpy as jnp
from jax import lax
from jax.experimental import pallas as pl
from jax.experimental.pallas import tpu as pltpu
from jax.experimental.pallas import tpu_sc as plsc

BATCH = 32
SEQ = 1024
EMB = 768
LANES = 16

NUM_CORES = 2
NUM_SUBCORES = 16
NUM_WORKERS = NUM_CORES * NUM_SUBCORES
S_PER_W = SEQ // NUM_WORKERS
VECS_PER_ROW = EMB // LANES


def _body(x_hbm, tok_hbm, pos_hbm, out_hbm, idx_v, rows_v, pos_v, sem):
  wid = lax.axis_index("s") * NUM_CORES + lax.axis_index("c")
  s_base = wid * S_PER_W

  pltpu.sync_copy(pos_hbm.at[pl.ds(s_base, S_PER_W)], pos_v)

  def batch_step(b, _):
    pltpu.sync_copy(x_hbm.at[b, pl.ds(s_base, S_PER_W)], idx_v)
    pltpu.async_copy(tok_hbm.at[idx_v], rows_v, sem).wait()

    def row_add(r, _):
      def vec_add(j, _):
        sl = pl.ds(j * LANES, LANES)
        rows_v[r, sl] = rows_v[r, sl] + pos_v[r, sl]
        return ()
      lax.fori_loop(0, VECS_PER_ROW, vec_add, ())
      return ()
    lax.fori_loop(0, S_PER_W, row_add, ())

    pltpu.sync_copy(rows_v, out_hbm.at[b, pl.ds(s_base, S_PER_W)])
    return ()

  lax.fori_loop(0, BATCH, batch_step, ())


@jax.jit
def kernel(x, tok_emb, pos_emb):
  mesh = plsc.VectorSubcoreMesh(
      core_axis_name="c", subcore_axis_name="s",
      num_cores=NUM_CORES, num_subcores=NUM_SUBCORES)
  f = pl.kernel(
      _body,
      out_type=jax.ShapeDtypeStruct((BATCH, SEQ, EMB), jnp.float32),
      mesh=mesh,
      scratch_types=[
          pltpu.VMEM((S_PER_W,), jnp.int32),
          pltpu.VMEM((S_PER_W, EMB), jnp.float32),
          pltpu.VMEM((S_PER_W, EMB), jnp.float32),
          pltpu.SemaphoreType.DMA,
      ],
  )
  return f(x.astype(jnp.int32), tok_emb, pos_emb)


# speedup vs baseline: 2.7449x; 2.7449x over previous
import jax
import jax.numpy as jnp
from jax import lax
from jax.experimental import pallas as pl
from jax.experimental.pallas import tpu as pltpu
from jax.experimental.pallas import tpu_sc as plsc

BATCH = 32
SEQ = 1024
EMB = 768
LANES = 16

NUM_CORES = 2
NUM_SUBCORES = 16
NUM_WORKERS = NUM_CORES * NUM_SUBCORES
S_PER_W = SEQ // NUM_WORKERS
VECS_PER_ROW = EMB // LANES


def _body(x_hbm, tok_hbm, pos_hbm, out_hbm,
          idx_a, idx_b, rows_a, rows_b, pos_v,
          gsem_a, gsem_b, osem_a, osem_b):
  wid = lax.axis_index("s") * NUM_CORES + lax.axis_index("c")
  s_base = wid * S_PER_W

  pltpu.sync_copy(pos_hbm.at[pl.ds(s_base, S_PER_W)], pos_v)

  def add_pos(rows_v):
    @plsc.parallel_loop(0, S_PER_W, 1, unroll=2)
    def _(r):
      for j in range(VECS_PER_ROW):
        sl = pl.ds(j * LANES, LANES)
        rows_v[r, sl] = rows_v[r, sl] + pos_v[r, sl]

  def start_gather(b, idx_v, rows_v, gsem):
    pltpu.sync_copy(x_hbm.at[b, pl.ds(s_base, S_PER_W)], idx_v)
    return pltpu.async_copy(tok_hbm.at[idx_v], rows_v, gsem)

  def out_copy(b, rows_v, osem):
    return pltpu.make_async_copy(
        rows_v, out_hbm.at[b, pl.ds(s_base, S_PER_W)], osem)

  start_gather(0, idx_a, rows_a, gsem_a)

  def step(g, _):
    @pl.when(g > 0)
    def _():
      out_copy(g - 1, rows_b, osem_b).wait()
    start_gather(g + 1, idx_b, rows_b, gsem_b)
    pltpu.make_async_copy(tok_hbm.at[idx_a], rows_a, gsem_a).wait()
    add_pos(rows_a)
    out_copy(g, rows_a, osem_a).start()

    @pl.when(g + 2 < BATCH)
    def _():
      out_copy(g, rows_a, osem_a).wait()
      start_gather(g + 2, idx_a, rows_a, gsem_a)
    pltpu.make_async_copy(tok_hbm.at[idx_b], rows_b, gsem_b).wait()
    add_pos(rows_b)
    out_copy(g + 1, rows_b, osem_b).start()
    return ()

  lax.fori_loop(0, BATCH // 2, lambda i, c: step(i * 2, c), (),
                unroll=False)

  out_copy(BATCH - 2, rows_a, osem_a).wait()
  out_copy(BATCH - 1, rows_b, osem_b).wait()


@jax.jit
def kernel(x, tok_emb, pos_emb):
  mesh = plsc.VectorSubcoreMesh(
      core_axis_name="c", subcore_axis_name="s",
      num_cores=NUM_CORES, num_subcores=NUM_SUBCORES)
  f = pl.kernel(
      _body,
      out_type=jax.ShapeDtypeStruct((BATCH, SEQ, EMB), jnp.float32),
      mesh=mesh,
      scratch_types=[
          pltpu.VMEM((S_PER_W,), jnp.int32),
          pltpu.VMEM((S_PER_W,), jnp.int32),
          pltpu.VMEM((S_PER_W, EMB), jnp.float32),
          pltpu.VMEM((S_PER_W, EMB), jnp.float32),
          pltpu.VMEM((S_PER_W, EMB), jnp.float32),
          pltpu.SemaphoreType.DMA,
          pltpu.SemaphoreType.DMA,
          pltpu.SemaphoreType.DMA,
          pltpu.SemaphoreType.DMA,
      ],
  )
  return f(x.astype(jnp.int32), tok_emb, pos_emb)


# speedup vs baseline: 3.0787x; 1.1216x over previous
import jax
import jax.numpy as jnp
from jax import lax
from jax.experimental import pallas as pl
from jax.experimental.pallas import tpu as pltpu
from jax.experimental.pallas import tpu_sc as plsc

BATCH = 32
SEQ = 1024
EMB = 768
LANES = 16

NUM_CORES = 2
NUM_SUBCORES = 16
NUM_WORKERS = NUM_CORES * NUM_SUBCORES
S_PER_W = SEQ // NUM_WORKERS
VECS_PER_ROW = EMB // LANES


def _body(xt_hbm, tok_hbm, pos_hbm, out_hbm,
          idx2d, rows_a, rows_b, pos_v,
          gsem_a, gsem_b, osem_a, osem_b):
  wid = lax.axis_index("s") * NUM_CORES + lax.axis_index("c")
  s_base = wid * S_PER_W

  pltpu.sync_copy(xt_hbm.at[pl.ds(s_base, S_PER_W)], idx2d)
  pltpu.sync_copy(pos_hbm.at[pl.ds(s_base, S_PER_W)], pos_v)

  def add_pos(s_local, rows_v):
    def col(j, _):
      sl = pl.ds(j * LANES, LANES)
      pvec = pos_v[s_local, sl]

      @plsc.parallel_loop(0, BATCH, 1, unroll=4)
      def _(r):
        rows_v[r, sl] = rows_v[r, sl] + pvec
      return ()
    lax.fori_loop(0, VECS_PER_ROW, col, ())

  def start_gather(s_local, rows_v, gsem):
    return pltpu.async_copy(tok_hbm.at[idx2d.at[s_local]], rows_v, gsem)

  def out_copy(s_local, rows_v, osem):
    return pltpu.make_async_copy(
        rows_v, out_hbm.at[:, s_base + s_local, :], osem)

  start_gather(0, rows_a, gsem_a)

  def step(g, _):
    @pl.when(g > 0)
    def _():
      out_copy(g - 1, rows_b, osem_b).wait()
    start_gather(g + 1, rows_b, gsem_b)
    pltpu.make_async_copy(tok_hbm.at[idx2d.at[g]], rows_a, gsem_a).wait()
    add_pos(g, rows_a)
    out_copy(g, rows_a, osem_a).start()

    @pl.when(g + 2 < S_PER_W)
    def _():
      out_copy(g, rows_a, osem_a).wait()
      start_gather(g + 2, rows_a, gsem_a)
    pltpu.make_async_copy(tok_hbm.at[idx2d.at[g + 1]], rows_b, gsem_b).wait()
    add_pos(g + 1, rows_b)
    out_copy(g + 1, rows_b, osem_b).start()
    return ()

  lax.fori_loop(0, S_PER_W // 2, lambda i, c: step(i * 2, c), (),
                unroll=False)

  out_copy(S_PER_W - 2, rows_a, osem_a).wait()
  out_copy(S_PER_W - 1, rows_b, osem_b).wait()


@jax.jit
def kernel(x, tok_emb, pos_emb):
  mesh = plsc.VectorSubcoreMesh(
      core_axis_name="c", subcore_axis_name="s",
      num_cores=NUM_CORES, num_subcores=NUM_SUBCORES)
  f = pl.kernel(
      _body,
      out_type=jax.ShapeDtypeStruct((BATCH, SEQ, EMB), jnp.float32),
      mesh=mesh,
      scratch_types=[
          pltpu.VMEM((S_PER_W, BATCH), jnp.int32),
          pltpu.VMEM((BATCH, EMB), jnp.float32),
          pltpu.VMEM((BATCH, EMB), jnp.float32),
          pltpu.VMEM((S_PER_W, EMB), jnp.float32),
          pltpu.SemaphoreType.DMA,
          pltpu.SemaphoreType.DMA,
          pltpu.SemaphoreType.DMA,
          pltpu.SemaphoreType.DMA,
      ],
  )
  xt = jnp.swapaxes(x.astype(jnp.int32), 0, 1)
  return f(xt, tok_emb, pos_emb)


# speedup vs baseline: 3.6222x; 1.1765x over previous
import jax
import jax.numpy as jnp
from jax import lax
from jax.experimental import pallas as pl
from jax.experimental.pallas import tpu as pltpu
from jax.experimental.pallas import tpu_sc as plsc

BATCH = 32
SEQ = 1024
EMB = 768
LANES = 16

NUM_CORES = 2
NUM_SUBCORES = 16
NUM_WORKERS = NUM_CORES * NUM_SUBCORES
S_PER_W = SEQ // NUM_WORKERS
VECS_PER_ROW = EMB // LANES


def _body(xt_hbm, oidx_hbm, tok_hbm, pos_hbm, out_hbm,
          idx2d, oidx2d, rows_a, rows_b, pos_v,
          gsem_a, gsem_b, osem_a, osem_b):
  wid = lax.axis_index("s") * NUM_CORES + lax.axis_index("c")
  s_base = wid * S_PER_W

  pltpu.sync_copy(xt_hbm.at[pl.ds(s_base, S_PER_W)], idx2d)
  pltpu.sync_copy(oidx_hbm.at[pl.ds(s_base, S_PER_W)], oidx2d)
  pltpu.sync_copy(pos_hbm.at[pl.ds(s_base, S_PER_W)], pos_v)

  def add_pos(s_local, rows_v):
    @plsc.parallel_loop(0, VECS_PER_ROW, 1)
    def _(j):
      sl = pl.ds(j * LANES, LANES)
      pvec = pos_v[s_local, sl]
      for r in range(BATCH):
        rows_v[r, sl] = rows_v[r, sl] + pvec

  def start_gather(s_local, rows_v, gsem):
    return pltpu.async_copy(tok_hbm.at[idx2d.at[s_local]], rows_v, gsem)

  def out_copy(s_local, rows_v, osem):
    return pltpu.make_async_copy(
        rows_v, out_hbm.at[oidx2d.at[s_local]], osem)

  start_gather(0, rows_a, gsem_a)

  def step(g, _):
    @pl.when(g > 0)
    def _():
      out_copy(g - 1, rows_b, osem_b).wait()
    start_gather(g + 1, rows_b, gsem_b)
    pltpu.make_async_copy(tok_hbm.at[idx2d.at[g]], rows_a, gsem_a).wait()
    add_pos(g, rows_a)
    out_copy(g, rows_a, osem_a).start()

    @pl.when(g + 2 < S_PER_W)
    def _():
      out_copy(g, rows_a, osem_a).wait()
      start_gather(g + 2, rows_a, gsem_a)
    pltpu.make_async_copy(tok_hbm.at[idx2d.at[g + 1]], rows_b, gsem_b).wait()
    add_pos(g + 1, rows_b)
    out_copy(g + 1, rows_b, osem_b).start()
    return ()

  lax.fori_loop(0, S_PER_W // 2, lambda i, c: step(i * 2, c), (),
                unroll=False)

  out_copy(S_PER_W - 2, rows_a, osem_a).wait()
  out_copy(S_PER_W - 1, rows_b, osem_b).wait()


@jax.jit
def kernel(x, tok_emb, pos_emb):
  mesh = plsc.VectorSubcoreMesh(
      core_axis_name="c", subcore_axis_name="s",
      num_cores=NUM_CORES, num_subcores=NUM_SUBCORES)
  f = pl.kernel(
      _body,
      out_type=jax.ShapeDtypeStruct((BATCH * SEQ, EMB), jnp.float32),
      mesh=mesh,
      scratch_types=[
          pltpu.VMEM((S_PER_W, BATCH), jnp.int32),
          pltpu.VMEM((S_PER_W, BATCH), jnp.int32),
          pltpu.VMEM((BATCH, EMB), jnp.float32),
          pltpu.VMEM((BATCH, EMB), jnp.float32),
          pltpu.VMEM((S_PER_W, EMB), jnp.float32),
          pltpu.SemaphoreType.DMA,
          pltpu.SemaphoreType.DMA,
          pltpu.SemaphoreType.DMA,
          pltpu.SemaphoreType.DMA,
      ],
  )
  xt = jnp.swapaxes(x.astype(jnp.int32), 0, 1)
  oidx = (jnp.arange(BATCH, dtype=jnp.int32)[None, :] * SEQ
          + jnp.arange(SEQ, dtype=jnp.int32)[:, None])
  out2d = f(xt, oidx, tok_emb, pos_emb)
  return out2d.reshape(BATCH, SEQ, EMB)


# speedup vs baseline: 3.6941x; 1.0199x over previous
import jax
import jax.numpy as jnp
from jax import lax
from jax.experimental import pallas as pl
from jax.experimental.pallas import tpu as pltpu
from jax.experimental.pallas import tpu_sc as plsc

BATCH = 32
SEQ = 1024
EMB = 768
LANES = 16

NUM_CORES = 2
NUM_SUBCORES = 16
NUM_WORKERS = NUM_CORES * NUM_SUBCORES
S_PER_W = SEQ // NUM_WORKERS
VECS_PER_ROW = EMB // LANES
NBUF = 4


def _body(xt_hbm, oidx_hbm, tok_hbm, pos_hbm, out_hbm,
          idx2d, oidx2d, pos_v, rows, gsems, osems):
  wid = lax.axis_index("s") * NUM_CORES + lax.axis_index("c")
  s_base = wid * S_PER_W

  pltpu.sync_copy(xt_hbm.at[pl.ds(s_base, S_PER_W)], idx2d)
  pltpu.sync_copy(oidx_hbm.at[pl.ds(s_base, S_PER_W)], oidx2d)
  pltpu.sync_copy(pos_hbm.at[pl.ds(s_base, S_PER_W)], pos_v)

  def add_pos(s_local, rows_v):
    @plsc.parallel_loop(0, VECS_PER_ROW, 1)
    def _(j):
      sl = pl.ds(j * LANES, LANES)
      pvec = pos_v[s_local, sl]
      for r in range(BATCH):
        rows_v[r, sl] = rows_v[r, sl] + pvec

  def gather(s_local, k):
    return pltpu.make_async_copy(
        tok_hbm.at[idx2d.at[s_local]], rows[k], gsems[k])

  def out_copy(s_local, k):
    return pltpu.make_async_copy(
        rows[k], out_hbm.at[oidx2d.at[s_local]], osems[k])

  gather(0, 0).start()
  gather(1, 1).start()

  def group(g, _):
    for k in range(NBUF):
      s = g * NBUF + k
      kp = (k + 2) % NBUF
      @pl.when(s >= 2)
      def _():
        out_copy(s - 2, kp).wait()
      @pl.when(s + 2 < S_PER_W)
      def _():
        gather(s + 2, kp).start()
      gather(s, k).wait()
      add_pos(s, rows[k])
      out_copy(s, k).start()
    return ()

  lax.fori_loop(0, S_PER_W // NBUF, group, (), unroll=False)

  out_copy(S_PER_W - 2, NBUF - 2).wait()
  out_copy(S_PER_W - 1, NBUF - 1).wait()


@jax.jit
def kernel(x, tok_emb, pos_emb):
  mesh = plsc.VectorSubcoreMesh(
      core_axis_name="c", subcore_axis_name="s",
      num_cores=NUM_CORES, num_subcores=NUM_SUBCORES)
  f = pl.kernel(
      _body,
      out_type=jax.ShapeDtypeStruct((BATCH * SEQ, EMB), jnp.float32),
      mesh=mesh,
      scratch_types=[
          pltpu.VMEM((S_PER_W, BATCH), jnp.int32),
          pltpu.VMEM((S_PER_W, BATCH), jnp.int32),
          pltpu.VMEM((S_PER_W, EMB), jnp.float32),
          [pltpu.VMEM((BATCH, EMB), jnp.float32) for _ in range(NBUF)],
          [pltpu.SemaphoreType.DMA for _ in range(NBUF)],
          [pltpu.SemaphoreType.DMA for _ in range(NBUF)],
      ],
  )
  xt = jnp.swapaxes(x.astype(jnp.int32), 0, 1)
  oidx = (jnp.arange(BATCH, dtype=jnp.int32)[None, :] * SEQ
          + jnp.arange(SEQ, dtype=jnp.int32)[:, None])
  out2d = f(xt, oidx, tok_emb, pos_emb)
  return out2d.reshape(BATCH, SEQ, EMB)
